# baseline (speedup 1.0000x reference)
import functools

import jax
import jax.numpy as jnp
from jax.experimental import pallas as pl
from jax.experimental.pallas import tpu as pltpu

SD, AD, TRAJ, BR = 256, 64, 1024, 32
N0 = 64 * 1024
NI = TRAJ * BR
R0 = 4096
RI = 4096
F32 = jnp.float32


def _val0_body(d_ref, pol_ref, sE_ref, wa_ref, wv_ref, n_ref, v_ref):
    cact = pol_ref[...] + 0.1 * n_ref[...]
    z = sE_ref[...] + jnp.dot(cact, wa_ref[...], preferred_element_type=F32)
    cns = jnp.tanh(z)
    proj = jnp.dot(cns, wv_ref[...], preferred_element_type=F32)
    v = (cns[:, 0:1] - cns[:, 1:2] - d_ref[0, 0]) + proj
    v_ref[...] = v[:, 0]


def _vali_body(d_ref, E_ref, P_ref, wa_ref, wv_ref, n_ref, v_ref):
    par = RI // BR
    Pr = jnp.broadcast_to(P_ref[...][:, None, :], (par, BR, AD)).reshape(RI, AD)
    Er = jnp.broadcast_to(E_ref[...][:, None, :], (par, BR, SD)).reshape(RI, SD)
    cact = Pr + 0.1 * n_ref[...]
    cns = jnp.tanh(Er + jnp.dot(cact, wa_ref[...], preferred_element_type=F32))
    proj = jnp.dot(cns, wv_ref[...], preferred_element_type=F32)
    v = (cns[:, 0:1] - cns[:, 1:2] - d_ref[0, 0]) + proj
    v_ref[...] = v[:, 0]


def _gstate_body(idx_ref, E_ref, P_ref, n_ref, wa_ref, we_ref, wp_ref,
                 Eo_ref, Po_ref, gE_s, gP_s, gn_s, pdiv):
    def body(r, carry):
        j = idx_ref[r]
        p = j // pdiv
        gn_s[pl.ds(r, 1), :] = n_ref[pl.ds(j, 1), :]
        gE_s[pl.ds(r, 1), :] = E_ref[pl.ds(p, 1), :]
        gP_s[pl.ds(r, 1), :] = P_ref[pl.ds(p, 1), :]
        return carry

    jax.lax.fori_loop(0, TRAJ, body, 0)
    cact = gP_s[...] + 0.1 * gn_s[...]
    S = jnp.tanh(gE_s[...] + jnp.dot(cact, wa_ref[...], preferred_element_type=F32))
    Eo_ref[...] = jnp.dot(S, we_ref[...], preferred_element_type=F32)
    Po_ref[...] = jnp.tanh(jnp.dot(S, wp_ref[...], preferred_element_type=F32))


def _values0(diff0, pol0, sE, W_act, wv, noise0):
    return pl.pallas_call(
        _val0_body,
        grid=(N0 // R0,),
        in_specs=[
            pl.BlockSpec(memory_space=pltpu.SMEM),
            pl.BlockSpec((1, AD), lambda i: (0, 0)),
            pl.BlockSpec((1, SD), lambda i: (0, 0)),
            pl.BlockSpec((AD, SD), lambda i: (0, 0)),
            pl.BlockSpec((SD, 1), lambda i: (0, 0)),
            pl.BlockSpec((R0, AD), lambda i: (i, 0)),
        ],
        out_specs=pl.BlockSpec((R0,), lambda i: (i,)),
        out_shape=jax.ShapeDtypeStruct((N0,), F32),
    )(diff0, pol0, sE, W_act, wv, noise0)


def _valuesi(diff0, E, P, W_act, wv, noise_i):
    par = RI // BR
    return pl.pallas_call(
        _vali_body,
        grid=(NI // RI,),
        in_specs=[
            pl.BlockSpec(memory_space=pltpu.SMEM),
            pl.BlockSpec((par, SD), lambda i: (i, 0)),
            pl.BlockSpec((par, AD), lambda i: (i, 0)),
            pl.BlockSpec((AD, SD), lambda i: (0, 0)),
            pl.BlockSpec((SD, 1), lambda i: (0, 0)),
            pl.BlockSpec((RI, AD), lambda i: (i, 0)),
        ],
        out_specs=pl.BlockSpec((RI,), lambda i: (i,)),
        out_shape=jax.ShapeDtypeStruct((NI,), F32),
    )(diff0, E, P, W_act, wv, noise_i)


def _gstate(idx, E, P, noise_i, W_act, W_evolve, W_policy, pdiv):
    nn = noise_i.shape[0]
    ne = E.shape[0]
    return pl.pallas_call(
        functools.partial(_gstate_body, pdiv=pdiv),
        in_specs=[
            pl.BlockSpec(memory_space=pltpu.SMEM),
            pl.BlockSpec((ne, SD), lambda: (0, 0)),
            pl.BlockSpec((ne, AD), lambda: (0, 0)),
            pl.BlockSpec((nn, AD), lambda: (0, 0)),
            pl.BlockSpec((AD, SD), lambda: (0, 0)),
            pl.BlockSpec((SD, SD), lambda: (0, 0)),
            pl.BlockSpec((SD, AD), lambda: (0, 0)),
        ],
        out_shape=(
            jax.ShapeDtypeStruct((TRAJ, SD), F32),
            jax.ShapeDtypeStruct((TRAJ, AD), F32),
        ),
        scratch_shapes=[
            pltpu.VMEM((TRAJ, SD), F32),
            pltpu.VMEM((TRAJ, AD), F32),
            pltpu.VMEM((TRAJ, AD), F32),
        ],
    )(idx, E, P, noise_i, W_act, W_evolve, W_policy)


def kernel(s_t, W_policy, W_evolve, W_act, w_val, noise0, noise):
    s0 = s_t.reshape(1, SD)
    diff0 = (s_t[0] - s_t[1]).reshape(1, 1)
    pol0 = jnp.tanh(s0 @ W_policy)
    sE = s0 @ W_evolve
    wv = w_val.reshape(SD, 1)

    v0 = _values0(diff0, pol0, sE, W_act, wv, noise0)
    _, idx0 = jax.lax.top_k(v0, TRAJ)
    chain = [idx0]

    gE = jnp.broadcast_to(sE, (TRAJ, SD))
    gP = jnp.broadcast_to(pol0, (TRAJ, AD))
    E, P = _gstate(idx0, gE, gP, noise0, W_act, W_evolve, W_policy, pdiv=N0)

    j3 = jnp.int32(0)
    for i in range(1, 4):
        v = _valuesi(diff0, E, P, W_act, wv, noise[i - 1])
        if i < 3:
            _, idx = jax.lax.top_k(v, TRAJ)
            chain.append(idx)
            E, P = _gstate(idx, E, P, noise[i - 1], W_act, W_evolve, W_policy, pdiv=BR)
        else:
            j3 = jnp.argmax(v)

    t2 = j3 // BR
    j2 = chain[2][t2]
    t1 = j2 // BR
    j1 = chain[1][t1]
    t0 = j1 // BR
    a = chain[0][t0]
    return pol0[0] + 0.1 * noise0[a]
